# Initial kernel scaffold; baseline (speedup 1.0000x reference)
import jax
import jax.numpy as jnp
from jax.experimental import pallas as pl


def kernel(x, edge_index, batch, W1_0, b1_0, W2_0, b2_0, g_0, be_0, W1_1, b1_1, W2_1, b2_1, g_1, be_1, W1_2, b1_2, W2_2, b2_2, g_2, be_2, Wp, bp):
    raise NotImplementedError("write your pallas kernel here")


# speedup vs baseline: 6.9400x; 6.9400x over previous
import functools

import jax
import jax.numpy as jnp
from jax import lax
from jax.experimental import pallas as pl
from jax.experimental.pallas import tpu as pltpu
from jax.experimental.pallas import tpu_sc as plsc

N = 50000
H = 64
HH = 32
G = 512
NC = 2
NS = 16
CH = 512
KSUB = 4
ACC_ROWS = 50048
ACC_PER_TILE = ACC_ROWS // NS
LAST_ROWS = N - 15 * ACC_PER_TILE
BLK = 2000
NB = N // BLK


def _sc_agg_body(epad, t2, srcb, dstb, zer, out, accum, sidx, didx, rows,
                 gsem, ssem):
    nrow = epad // 128
    tile_rows = nrow // NS
    iters = tile_rows // KSUB
    c = lax.axis_index("c")
    s = lax.axis_index("s")
    pltpu.sync_copy(zer.at[pl.ds(s * ACC_PER_TILE, ACC_PER_TILE)],
                    accum.at[pl.ds(s * ACC_PER_TILE, ACC_PER_TILE)])
    plsc.subcore_barrier()
    src_row0 = c * nrow + s * tile_rows
    dst_row0 = s * tile_rows

    def body(g, carry):
        pltpu.sync_copy(srcb.at[pl.ds(src_row0 + g * KSUB, KSUB)], sidx)
        pltpu.sync_copy(dstb.at[pl.ds(dst_row0 + g * KSUB, KSUB)], didx)
        hs = [pltpu.async_copy(t2.at[sidx.at[j]],
                               rows.at[pl.ds(j * 128, 128)], gsem)
              for j in range(KSUB)]
        for h in hs:
            h.wait()
        hs = [pltpu.async_copy(rows.at[pl.ds(j * 128, 128)],
                               accum.at[didx.at[j]], ssem, add=True)
              for j in range(KSUB)]
        for h in hs:
            h.wait()
        return carry

    lax.fori_loop(0, iters, body, 0)
    plsc.subcore_barrier()

    @pl.when(s < NS - 1)
    def _():
        pltpu.sync_copy(accum.at[pl.ds(s * ACC_PER_TILE, ACC_PER_TILE)],
                        out.at[pl.ds(c * N + s * ACC_PER_TILE, ACC_PER_TILE)])

    @pl.when(s == NS - 1)
    def _():
        pltpu.sync_copy(accum.at[pl.ds((NS - 1) * ACC_PER_TILE, LAST_ROWS)],
                        out.at[pl.ds(c * N + (NS - 1) * ACC_PER_TILE,
                                     LAST_ROWS)])


def _make_sc_agg(epad):
    return pl.kernel(
        functools.partial(_sc_agg_body, epad),
        out_type=jax.ShapeDtypeStruct((2 * N, HH), jnp.float32),
        mesh=plsc.VectorSubcoreMesh(core_axis_name="c", subcore_axis_name="s"),
        compiler_params=pltpu.CompilerParams(use_tc_tiling_on_sc=False),
        scratch_types=[
            pltpu.VMEM_SHARED((ACC_ROWS, HH), jnp.float32),
            pltpu.VMEM((KSUB, 128), jnp.int32),
            pltpu.VMEM((KSUB, 128), jnp.int32),
            pltpu.VMEM((CH, HH), jnp.float32),
            pltpu.SemaphoreType.DMA,
            pltpu.SemaphoreType.DMA,
        ],
    )


def _t0_body(x_ref, w_ref, out_ref):
    t = jnp.dot(x_ref[...], w_ref[...], preferred_element_type=jnp.float32)
    out_ref[0] = t[:, :HH]
    out_ref[1] = t[:, HH:]


def _k2_body(t_ref, a_ref, b1_ref, w2_ref, b2_ref, v_ref, st_ref, ssum, ssq):
    i = pl.program_id(0)
    tb = jnp.concatenate([t_ref[0], t_ref[1]], axis=1)
    ab = jnp.concatenate([a_ref[0], a_ref[1]], axis=1)
    u = jnp.maximum(tb + ab + b1_ref[...], 0.0)
    v = jnp.dot(u, w2_ref[...], preferred_element_type=jnp.float32) + b2_ref[...]
    v_ref[...] = v

    @pl.when(i == 0)
    def _():
        ssum[...] = jnp.zeros_like(ssum)
        ssq[...] = jnp.zeros_like(ssq)

    ssum[...] += jnp.sum(v, axis=0, keepdims=True)
    ssq[...] += jnp.sum(v * v, axis=0, keepdims=True)

    @pl.when(i == pl.num_programs(0) - 1)
    def _():
        st_ref[0:1] = ssum[...]
        st_ref[1:2] = ssq[...]


def _bn(v, st_ref, g_ref, be_ref):
    mean = st_ref[0:1] * (1.0 / N)
    var = st_ref[1:2] * (1.0 / N) - mean * mean
    scale = g_ref[...] * lax.rsqrt(var + 1e-5)
    shift = be_ref[...] - mean * scale
    return jnp.maximum(v * scale + shift, 0.0)


def _k3_body(v_ref, st_ref, g_ref, be_ref, w1n_ref, out_ref):
    h = _bn(v_ref[...], st_ref, g_ref, be_ref)
    t = jnp.dot(h, w1n_ref[...], preferred_element_type=jnp.float32)
    out_ref[0] = t[:, :HH]
    out_ref[1] = t[:, HH:]


def _k3p_body(v_ref, st_ref, g_ref, be_ref, batch_ref, ps_ref, cnt_ref,
              pacc, cacc):
    i = pl.program_id(0)
    h = _bn(v_ref[...], st_ref, g_ref, be_ref)
    ids = lax.broadcasted_iota(jnp.int32, (BLK, G), 1)
    oh = (batch_ref[0, 0][:, None] == ids).astype(jnp.float32)

    @pl.when(i == 0)
    def _():
        pacc[...] = jnp.zeros_like(pacc)
        cacc[...] = jnp.zeros_like(cacc)

    dn = (((0,), (0,)), ((), ()))
    pacc[...] += lax.dot_general(oh, h, dn, preferred_element_type=jnp.float32)
    cacc[...] += lax.dot_general(oh, jnp.ones((BLK, 1), jnp.float32), dn,
                                 preferred_element_type=jnp.float32)

    @pl.when(i == pl.num_programs(0) - 1)
    def _():
        ps_ref[...] = pacc[...]
        cnt_ref[...] = cacc[...]


def _fin_body(ps_ref, cnt_ref, wp_ref, bp_ref, out_ref):
    inv = 1.0 / jnp.maximum(cnt_ref[...], 1.0)
    pooled = ps_ref[...] * inv
    o = jnp.dot(pooled, wp_ref[...], preferred_element_type=jnp.float32)
    o = o + bp_ref[...]
    nrm = jnp.sqrt(jnp.sum(o * o, axis=1, keepdims=True))
    out_ref[...] = o / jnp.maximum(nrm, 1e-12)


def _spec(shape, imap):
    return pl.BlockSpec(shape, imap)


_t0 = pl.pallas_call(
    _t0_body,
    grid=(NB,),
    in_specs=[_spec((BLK, 128), lambda i: (i, 0)),
              _spec((128, H), lambda i: (0, 0))],
    out_specs=_spec((2, BLK, HH), lambda i: (0, i, 0)),
    out_shape=jax.ShapeDtypeStruct((2, N, HH), jnp.float32),
)

_k2 = pl.pallas_call(
    _k2_body,
    grid=(NB,),
    in_specs=[_spec((2, BLK, HH), lambda i: (0, i, 0)),
              _spec((2, BLK, HH), lambda i: (0, i, 0)),
              _spec((1, H), lambda i: (0, 0)),
              _spec((H, H), lambda i: (0, 0)),
              _spec((1, H), lambda i: (0, 0))],
    out_specs=[_spec((BLK, H), lambda i: (i, 0)),
               _spec((2, H), lambda i: (0, 0))],
    out_shape=[jax.ShapeDtypeStruct((N, H), jnp.float32),
               jax.ShapeDtypeStruct((2, H), jnp.float32)],
    scratch_shapes=[pltpu.VMEM((1, H), jnp.float32),
                    pltpu.VMEM((1, H), jnp.float32)],
)

_k3 = pl.pallas_call(
    _k3_body,
    grid=(NB,),
    in_specs=[_spec((BLK, H), lambda i: (i, 0)),
              _spec((2, H), lambda i: (0, 0)),
              _spec((1, H), lambda i: (0, 0)),
              _spec((1, H), lambda i: (0, 0)),
              _spec((H, H), lambda i: (0, 0))],
    out_specs=_spec((2, BLK, HH), lambda i: (0, i, 0)),
    out_shape=jax.ShapeDtypeStruct((2, N, HH), jnp.float32),
)

_k3p = pl.pallas_call(
    _k3p_body,
    grid=(NB,),
    in_specs=[_spec((BLK, H), lambda i: (i, 0)),
              _spec((2, H), lambda i: (0, 0)),
              _spec((1, H), lambda i: (0, 0)),
              _spec((1, H), lambda i: (0, 0)),
              _spec((1, 1, BLK), lambda i: (i, 0, 0))],
    out_specs=[_spec((G, H), lambda i: (0, 0)),
               _spec((G, 1), lambda i: (0, 0))],
    out_shape=[jax.ShapeDtypeStruct((G, H), jnp.float32),
               jax.ShapeDtypeStruct((G, 1), jnp.float32)],
    scratch_shapes=[pltpu.VMEM((G, H), jnp.float32),
                    pltpu.VMEM((G, 1), jnp.float32)],
)

_fin = pl.pallas_call(
    _fin_body,
    in_specs=[_spec((G, H), lambda: (0, 0)),
              _spec((G, 1), lambda: (0, 0)),
              _spec((H, H), lambda: (0, 0)),
              _spec((1, H), lambda: (0, 0))],
    out_specs=_spec((G, H), lambda: (0, 0)),
    out_shape=jax.ShapeDtypeStruct((G, H), jnp.float32),
)


def kernel(x, edge_index, batch, W1_0, b1_0, W2_0, b2_0, g_0, be_0,
           W1_1, b1_1, W2_1, b2_1, g_1, be_1, W1_2, b1_2, W2_2, b2_2,
           g_2, be_2, Wp, bp):
    E = edge_index.shape[1]
    chunk = NS * CH
    epad = -(-E // chunk) * chunk
    pad = epad - E
    src = edge_index[0]
    dst = edge_index[1]
    ar = jnp.arange(pad, dtype=jnp.int32)
    src_p = jnp.concatenate([src, ar % N])
    dst_p = jnp.concatenate([dst, N + ar % (ACC_ROWS - N)])
    srcb = jnp.concatenate([src_p, src_p + N]).reshape(2 * (epad // 128), 128)
    dstb = dst_p.reshape(epad // 128, 128)
    zer = jnp.zeros((ACC_ROWS, HH), jnp.float32)
    sc_agg = _make_sc_agg(epad)

    b1s = (b1_0.reshape(1, H), b1_1.reshape(1, H), b1_2.reshape(1, H))
    b2s = (b2_0.reshape(1, H), b2_1.reshape(1, H), b2_2.reshape(1, H))
    gs = (g_0.reshape(1, H), g_1.reshape(1, H), g_2.reshape(1, H))
    bes = (be_0.reshape(1, H), be_1.reshape(1, H), be_2.reshape(1, H))
    W2s = (W2_0, W2_1, W2_2)
    W1n = (W1_1, W1_2)

    t = _t0(x, W1_0)
    for l in range(3):
        agg2 = sc_agg(t.reshape(2 * N, HH), srcb, dstb, zer)
        agg = agg2.reshape(2, N, HH)
        v, st = _k2(t, agg, b1s[l], W2s[l], b2s[l])
        if l < 2:
            t = _k3(v, st, gs[l], bes[l], W1n[l])
        else:
            ps, cnt = _k3p(v, st, gs[l], bes[l], batch.reshape(NB, 1, BLK))
    return _fin(ps, cnt, Wp, bp.reshape(1, H))


# speedup vs baseline: 8.2743x; 1.1923x over previous
import functools

import jax
import jax.numpy as jnp
from jax import lax
from jax.experimental import pallas as pl
from jax.experimental.pallas import tpu as pltpu
from jax.experimental.pallas import tpu_sc as plsc

N = 50000
H = 64
HH = 32
G = 512
NC = 2
NS = 16
CH = 384
KSUB = 3
ACC_ROWS = 50048
ACC_PER_TILE = ACC_ROWS // NS
LAST_ROWS = N - 15 * ACC_PER_TILE
BLK = 2000
NB = N // BLK


def _sc_agg_body(epad, t2, srcb, dstb, zer, out, accum, sidx, didx, rows,
                 gsems, ssems):
    nrow = epad // 128
    tile_rows = nrow // NS
    iters = tile_rows // KSUB
    c = lax.axis_index("c")
    s = lax.axis_index("s")
    pltpu.sync_copy(zer.at[pl.ds(s * ACC_PER_TILE, ACC_PER_TILE)],
                    accum.at[pl.ds(s * ACC_PER_TILE, ACC_PER_TILE)])
    plsc.subcore_barrier()
    src_row0 = c * nrow + s * tile_rows
    dst_row0 = s * tile_rows

    def idx_load(b, g):
        pltpu.sync_copy(srcb.at[pl.ds(src_row0 + g * KSUB, KSUB)], sidx.at[b])
        pltpu.sync_copy(dstb.at[pl.ds(dst_row0 + g * KSUB, KSUB)], didx.at[b])

    def gather(b, j):
        return (t2.at[sidx.at[b, j]], rows.at[b, pl.ds(j * 128, 128)],
                gsems.at[b])

    def scatter(b, j):
        return (rows.at[b, pl.ds(j * 128, 128)], accum.at[didx.at[b, j]],
                ssems.at[b])

    def fire_gathers(b):
        for j in range(KSUB):
            pltpu.async_copy(*gather(b, j))

    def wait_gathers(b):
        for j in range(KSUB):
            pltpu.make_async_copy(*gather(b, j)).wait()

    def fire_scatters(b):
        for j in range(KSUB):
            pltpu.async_copy(*scatter(b, j), add=True)

    def wait_scatters(b):
        for j in range(KSUB):
            pltpu.make_async_copy(*scatter(b, j)).wait()

    idx_load(0, 0)
    fire_gathers(0)

    def body(g, carry):
        b = g % 2
        nb = 1 - b

        @pl.when(g >= 1)
        def _():
            wait_scatters(nb)

        @pl.when(g + 1 < iters)
        def _():
            idx_load(nb, g + 1)
            fire_gathers(nb)

        wait_gathers(b)
        fire_scatters(b)
        return carry

    lax.fori_loop(0, iters, body, 0)
    wait_scatters((iters - 1) % 2)
    plsc.subcore_barrier()

    @pl.when(s < NS - 1)
    def _():
        pltpu.sync_copy(accum.at[pl.ds(s * ACC_PER_TILE, ACC_PER_TILE)],
                        out.at[pl.ds(c * N + s * ACC_PER_TILE, ACC_PER_TILE)])

    @pl.when(s == NS - 1)
    def _():
        pltpu.sync_copy(accum.at[pl.ds((NS - 1) * ACC_PER_TILE, LAST_ROWS)],
                        out.at[pl.ds(c * N + (NS - 1) * ACC_PER_TILE,
                                     LAST_ROWS)])


def _make_sc_agg(epad):
    return pl.kernel(
        functools.partial(_sc_agg_body, epad),
        out_type=jax.ShapeDtypeStruct((2 * N, HH), jnp.float32),
        mesh=plsc.VectorSubcoreMesh(core_axis_name="c", subcore_axis_name="s"),
        compiler_params=pltpu.CompilerParams(use_tc_tiling_on_sc=False),
        scratch_types=[
            pltpu.VMEM_SHARED((ACC_ROWS, HH), jnp.float32),
            pltpu.VMEM((2, KSUB, 128), jnp.int32),
            pltpu.VMEM((2, KSUB, 128), jnp.int32),
            pltpu.VMEM((2, CH, HH), jnp.float32),
            pltpu.SemaphoreType.DMA((2,)),
            pltpu.SemaphoreType.DMA((2,)),
        ],
    )


def _t0_body(x_ref, w_ref, out_ref):
    t = jnp.dot(x_ref[...], w_ref[...], preferred_element_type=jnp.float32)
    out_ref[0] = t[:, :HH]
    out_ref[1] = t[:, HH:]


def _k2_body(t_ref, a_ref, b1_ref, w2_ref, b2_ref, v_ref, st_ref, ssum, ssq):
    i = pl.program_id(0)
    tb = jnp.concatenate([t_ref[0], t_ref[1]], axis=1)
    ab = jnp.concatenate([a_ref[0], a_ref[1]], axis=1)
    u = jnp.maximum(tb + ab + b1_ref[...], 0.0)
    v = jnp.dot(u, w2_ref[...], preferred_element_type=jnp.float32) + b2_ref[...]
    v_ref[...] = v

    @pl.when(i == 0)
    def _():
        ssum[...] = jnp.zeros_like(ssum)
        ssq[...] = jnp.zeros_like(ssq)

    ssum[...] += jnp.sum(v, axis=0, keepdims=True)
    ssq[...] += jnp.sum(v * v, axis=0, keepdims=True)

    @pl.when(i == pl.num_programs(0) - 1)
    def _():
        st_ref[0:1] = ssum[...]
        st_ref[1:2] = ssq[...]


def _bn(v, st_ref, g_ref, be_ref):
    mean = st_ref[0:1] * (1.0 / N)
    var = st_ref[1:2] * (1.0 / N) - mean * mean
    scale = g_ref[...] * lax.rsqrt(var + 1e-5)
    shift = be_ref[...] - mean * scale
    return jnp.maximum(v * scale + shift, 0.0)


def _k3_body(v_ref, st_ref, g_ref, be_ref, w1n_ref, out_ref):
    h = _bn(v_ref[...], st_ref, g_ref, be_ref)
    t = jnp.dot(h, w1n_ref[...], preferred_element_type=jnp.float32)
    out_ref[0] = t[:, :HH]
    out_ref[1] = t[:, HH:]


def _k3p_body(v_ref, st_ref, g_ref, be_ref, batch_ref, ps_ref, cnt_ref,
              pacc, cacc):
    i = pl.program_id(0)
    h = _bn(v_ref[...], st_ref, g_ref, be_ref)
    ids = lax.broadcasted_iota(jnp.int32, (BLK, G), 1)
    oh = (batch_ref[0, 0][:, None] == ids).astype(jnp.float32)

    @pl.when(i == 0)
    def _():
        pacc[...] = jnp.zeros_like(pacc)
        cacc[...] = jnp.zeros_like(cacc)

    dn = (((0,), (0,)), ((), ()))
    pacc[...] += lax.dot_general(oh, h, dn, preferred_element_type=jnp.float32)
    cacc[...] += lax.dot_general(oh, jnp.ones((BLK, 1), jnp.float32), dn,
                                 preferred_element_type=jnp.float32)

    @pl.when(i == pl.num_programs(0) - 1)
    def _():
        ps_ref[...] = pacc[...]
        cnt_ref[...] = cacc[...]


def _fin_body(ps_ref, cnt_ref, wp_ref, bp_ref, out_ref):
    inv = 1.0 / jnp.maximum(cnt_ref[...], 1.0)
    pooled = ps_ref[...] * inv
    o = jnp.dot(pooled, wp_ref[...], preferred_element_type=jnp.float32)
    o = o + bp_ref[...]
    nrm = jnp.sqrt(jnp.sum(o * o, axis=1, keepdims=True))
    out_ref[...] = o / jnp.maximum(nrm, 1e-12)


def _spec(shape, imap):
    return pl.BlockSpec(shape, imap)


_t0 = pl.pallas_call(
    _t0_body,
    grid=(NB,),
    in_specs=[_spec((BLK, 128), lambda i: (i, 0)),
              _spec((128, H), lambda i: (0, 0))],
    out_specs=_spec((2, BLK, HH), lambda i: (0, i, 0)),
    out_shape=jax.ShapeDtypeStruct((2, N, HH), jnp.float32),
)

_k2 = pl.pallas_call(
    _k2_body,
    grid=(NB,),
    in_specs=[_spec((2, BLK, HH), lambda i: (0, i, 0)),
              _spec((2, BLK, HH), lambda i: (0, i, 0)),
              _spec((1, H), lambda i: (0, 0)),
              _spec((H, H), lambda i: (0, 0)),
              _spec((1, H), lambda i: (0, 0))],
    out_specs=[_spec((BLK, H), lambda i: (i, 0)),
               _spec((2, H), lambda i: (0, 0))],
    out_shape=[jax.ShapeDtypeStruct((N, H), jnp.float32),
               jax.ShapeDtypeStruct((2, H), jnp.float32)],
    scratch_shapes=[pltpu.VMEM((1, H), jnp.float32),
                    pltpu.VMEM((1, H), jnp.float32)],
)

_k3 = pl.pallas_call(
    _k3_body,
    grid=(NB,),
    in_specs=[_spec((BLK, H), lambda i: (i, 0)),
              _spec((2, H), lambda i: (0, 0)),
              _spec((1, H), lambda i: (0, 0)),
              _spec((1, H), lambda i: (0, 0)),
              _spec((H, H), lambda i: (0, 0))],
    out_specs=_spec((2, BLK, HH), lambda i: (0, i, 0)),
    out_shape=jax.ShapeDtypeStruct((2, N, HH), jnp.float32),
)

_k3p = pl.pallas_call(
    _k3p_body,
    grid=(NB,),
    in_specs=[_spec((BLK, H), lambda i: (i, 0)),
              _spec((2, H), lambda i: (0, 0)),
              _spec((1, H), lambda i: (0, 0)),
              _spec((1, H), lambda i: (0, 0)),
              _spec((1, 1, BLK), lambda i: (i, 0, 0))],
    out_specs=[_spec((G, H), lambda i: (0, 0)),
               _spec((G, 1), lambda i: (0, 0))],
    out_shape=[jax.ShapeDtypeStruct((G, H), jnp.float32),
               jax.ShapeDtypeStruct((G, 1), jnp.float32)],
    scratch_shapes=[pltpu.VMEM((G, H), jnp.float32),
                    pltpu.VMEM((G, 1), jnp.float32)],
)

_fin = pl.pallas_call(
    _fin_body,
    in_specs=[_spec((G, H), lambda: (0, 0)),
              _spec((G, 1), lambda: (0, 0)),
              _spec((H, H), lambda: (0, 0)),
              _spec((1, H), lambda: (0, 0))],
    out_specs=_spec((G, H), lambda: (0, 0)),
    out_shape=jax.ShapeDtypeStruct((G, H), jnp.float32),
)


def kernel(x, edge_index, batch, W1_0, b1_0, W2_0, b2_0, g_0, be_0,
           W1_1, b1_1, W2_1, b2_1, g_1, be_1, W1_2, b1_2, W2_2, b2_2,
           g_2, be_2, Wp, bp):
    E = edge_index.shape[1]
    chunk = 2 * NS * CH
    epad = -(-E // chunk) * chunk
    pad = epad - E
    src = edge_index[0]
    dst = edge_index[1]
    ar = jnp.arange(pad, dtype=jnp.int32)
    src_p = jnp.concatenate([src, ar % N])
    dst_p = jnp.concatenate([dst, N + ar % (ACC_ROWS - N)])
    srcb = jnp.concatenate([src_p, src_p + N]).reshape(2 * (epad // 128), 128)
    dstb = dst_p.reshape(epad // 128, 128)
    zer = jnp.zeros((ACC_ROWS, HH), jnp.float32)
    sc_agg = _make_sc_agg(epad)

    b1s = (b1_0.reshape(1, H), b1_1.reshape(1, H), b1_2.reshape(1, H))
    b2s = (b2_0.reshape(1, H), b2_1.reshape(1, H), b2_2.reshape(1, H))
    gs = (g_0.reshape(1, H), g_1.reshape(1, H), g_2.reshape(1, H))
    bes = (be_0.reshape(1, H), be_1.reshape(1, H), be_2.reshape(1, H))
    W2s = (W2_0, W2_1, W2_2)
    W1n = (W1_1, W1_2)

    t = _t0(x, W1_0)
    for l in range(3):
        agg2 = sc_agg(t.reshape(2 * N, HH), srcb, dstb, zer)
        agg = agg2.reshape(2, N, HH)
        v, st = _k2(t, agg, b1s[l], W2s[l], b2s[l])
        if l < 2:
            t = _k3(v, st, gs[l], bes[l], W1n[l])
        else:
            ps, cnt = _k3p(v, st, gs[l], bes[l], batch.reshape(NB, 1, BLK))
    return _fin(ps, cnt, Wp, bp.reshape(1, H))


# speedup vs baseline: 9.7159x; 1.1742x over previous
import functools

import jax
import jax.numpy as jnp
from jax import lax
from jax.experimental import pallas as pl
from jax.experimental.pallas import tpu as pltpu
from jax.experimental.pallas import tpu_sc as plsc

N = 50000
H = 64
HH = 32
G = 512
NC = 2
NS = 16
CH = 384
KSUB = 3
ACC_ROWS = 50048
ACC_PER_TILE = ACC_ROWS // NS
LAST_ROWS = N - 15 * ACC_PER_TILE
BLK = 2000
NB = N // BLK


def _sc_agg_body(epad, t3, srcdst, zer, out, accum, idx, rows, gsems, ssems):
    nrow = epad // 128
    tile_rows = nrow // NS
    iters = tile_rows // KSUB
    c = lax.axis_index("c")
    s = lax.axis_index("s")
    pltpu.sync_copy(zer.at[pl.ds(s * ACC_PER_TILE, ACC_PER_TILE)],
                    accum.at[pl.ds(s * ACC_PER_TILE, ACC_PER_TILE)])
    plsc.subcore_barrier()
    row0 = s * tile_rows

    def idx_load(b, g):
        pltpu.sync_copy(srcdst.at[pl.ds(row0 + g * KSUB, KSUB)], idx.at[b])

    def gather(b, j):
        return (t3.at[c].at[idx.at[b, j, 0]],
                rows.at[b, pl.ds(j * 128, 128)], gsems.at[b])

    def scatter(b, j):
        return (rows.at[b, pl.ds(j * 128, 128)], accum.at[idx.at[b, j, 1]],
                ssems.at[b])

    def fire_gathers(b):
        for j in range(KSUB):
            pltpu.async_copy(*gather(b, j))

    def fire_scatters(b):
        for j in range(KSUB):
            pltpu.make_async_copy(*gather(b, j)).wait()
            pltpu.async_copy(*scatter(b, j), add=True)

    def wait_scatters(b):
        for j in range(KSUB):
            pltpu.make_async_copy(*scatter(b, j)).wait()

    idx_load(0, 0)
    fire_gathers(0)

    def body(g, carry):
        b = g % 2
        nb = 1 - b

        @pl.when(g >= 1)
        def _():
            wait_scatters(nb)

        @pl.when(g + 1 < iters)
        def _():
            idx_load(nb, g + 1)
            fire_gathers(nb)

        fire_scatters(b)
        return carry

    lax.fori_loop(0, iters, body, 0)
    wait_scatters((iters - 1) % 2)
    plsc.subcore_barrier()

    @pl.when(s < NS - 1)
    def _():
        pltpu.sync_copy(accum.at[pl.ds(s * ACC_PER_TILE, ACC_PER_TILE)],
                        out.at[pl.ds(c * N + s * ACC_PER_TILE, ACC_PER_TILE)])

    @pl.when(s == NS - 1)
    def _():
        pltpu.sync_copy(accum.at[pl.ds((NS - 1) * ACC_PER_TILE, LAST_ROWS)],
                        out.at[pl.ds(c * N + (NS - 1) * ACC_PER_TILE,
                                     LAST_ROWS)])


def _make_sc_agg(epad):
    return pl.kernel(
        functools.partial(_sc_agg_body, epad),
        out_type=jax.ShapeDtypeStruct((2 * N, HH), jnp.float32),
        mesh=plsc.VectorSubcoreMesh(core_axis_name="c", subcore_axis_name="s"),
        compiler_params=pltpu.CompilerParams(use_tc_tiling_on_sc=False),
        scratch_types=[
            pltpu.VMEM_SHARED((ACC_ROWS, HH), jnp.float32),
            pltpu.VMEM((2, KSUB, 2, 128), jnp.int32),
            pltpu.VMEM((2, CH, HH), jnp.float32),
            pltpu.SemaphoreType.DMA((2,)),
            pltpu.SemaphoreType.DMA((2,)),
        ],
    )


def _t0_body(x_ref, w_ref, out_ref):
    t = jnp.dot(x_ref[...], w_ref[...], preferred_element_type=jnp.float32)
    out_ref[0] = t[:, :HH]
    out_ref[1] = t[:, HH:]


def _k2_body(t_ref, a_ref, b1_ref, w2_ref, b2_ref, v_ref, st_ref, ssum, ssq):
    i = pl.program_id(0)
    tb = jnp.concatenate([t_ref[0], t_ref[1]], axis=1)
    ab = jnp.concatenate([a_ref[0], a_ref[1]], axis=1)
    u = jnp.maximum(tb + ab + b1_ref[...], 0.0)
    v = jnp.dot(u, w2_ref[...], preferred_element_type=jnp.float32) + b2_ref[...]
    v_ref[...] = v

    @pl.when(i == 0)
    def _():
        ssum[...] = jnp.zeros_like(ssum)
        ssq[...] = jnp.zeros_like(ssq)

    ssum[...] += jnp.sum(v, axis=0, keepdims=True)
    ssq[...] += jnp.sum(v * v, axis=0, keepdims=True)

    @pl.when(i == pl.num_programs(0) - 1)
    def _():
        st_ref[0:1] = ssum[...]
        st_ref[1:2] = ssq[...]


def _bn(v, st_ref, g_ref, be_ref):
    mean = st_ref[0:1] * (1.0 / N)
    var = st_ref[1:2] * (1.0 / N) - mean * mean
    scale = g_ref[...] * lax.rsqrt(var + 1e-5)
    shift = be_ref[...] - mean * scale
    return jnp.maximum(v * scale + shift, 0.0)


def _k3_body(v_ref, st_ref, g_ref, be_ref, w1n_ref, out_ref):
    h = _bn(v_ref[...], st_ref, g_ref, be_ref)
    t = jnp.dot(h, w1n_ref[...], preferred_element_type=jnp.float32)
    out_ref[0] = t[:, :HH]
    out_ref[1] = t[:, HH:]


def _k3p_body(v_ref, st_ref, g_ref, be_ref, batch_ref, ps_ref, cnt_ref,
              pacc, cacc):
    i = pl.program_id(0)
    h = _bn(v_ref[...], st_ref, g_ref, be_ref)
    ids = lax.broadcasted_iota(jnp.int32, (BLK, G), 1)
    oh = (batch_ref[0, 0][:, None] == ids).astype(jnp.float32)

    @pl.when(i == 0)
    def _():
        pacc[...] = jnp.zeros_like(pacc)
        cacc[...] = jnp.zeros_like(cacc)

    dn = (((0,), (0,)), ((), ()))
    pacc[...] += lax.dot_general(oh, h, dn, preferred_element_type=jnp.float32)
    cacc[...] += lax.dot_general(oh, jnp.ones((BLK, 1), jnp.float32), dn,
                                 preferred_element_type=jnp.float32)

    @pl.when(i == pl.num_programs(0) - 1)
    def _():
        ps_ref[...] = pacc[...]
        cnt_ref[...] = cacc[...]


def _fin_body(ps_ref, cnt_ref, wp_ref, bp_ref, out_ref):
    inv = 1.0 / jnp.maximum(cnt_ref[...], 1.0)
    pooled = ps_ref[...] * inv
    o = jnp.dot(pooled, wp_ref[...], preferred_element_type=jnp.float32)
    o = o + bp_ref[...]
    nrm = jnp.sqrt(jnp.sum(o * o, axis=1, keepdims=True))
    out_ref[...] = o / jnp.maximum(nrm, 1e-12)


def _spec(shape, imap):
    return pl.BlockSpec(shape, imap)


_t0 = pl.pallas_call(
    _t0_body,
    grid=(NB,),
    in_specs=[_spec((BLK, 128), lambda i: (i, 0)),
              _spec((128, H), lambda i: (0, 0))],
    out_specs=_spec((2, BLK, HH), lambda i: (0, i, 0)),
    out_shape=jax.ShapeDtypeStruct((2, N, HH), jnp.float32),
)

_k2 = pl.pallas_call(
    _k2_body,
    grid=(NB,),
    in_specs=[_spec((2, BLK, HH), lambda i: (0, i, 0)),
              _spec((2, BLK, HH), lambda i: (0, i, 0)),
              _spec((1, H), lambda i: (0, 0)),
              _spec((H, H), lambda i: (0, 0)),
              _spec((1, H), lambda i: (0, 0))],
    out_specs=[_spec((BLK, H), lambda i: (i, 0)),
               _spec((2, H), lambda i: (0, 0))],
    out_shape=[jax.ShapeDtypeStruct((N, H), jnp.float32),
               jax.ShapeDtypeStruct((2, H), jnp.float32)],
    scratch_shapes=[pltpu.VMEM((1, H), jnp.float32),
                    pltpu.VMEM((1, H), jnp.float32)],
)

_k3 = pl.pallas_call(
    _k3_body,
    grid=(NB,),
    in_specs=[_spec((BLK, H), lambda i: (i, 0)),
              _spec((2, H), lambda i: (0, 0)),
              _spec((1, H), lambda i: (0, 0)),
              _spec((1, H), lambda i: (0, 0)),
              _spec((H, H), lambda i: (0, 0))],
    out_specs=_spec((2, BLK, HH), lambda i: (0, i, 0)),
    out_shape=jax.ShapeDtypeStruct((2, N, HH), jnp.float32),
)

_k3p = pl.pallas_call(
    _k3p_body,
    grid=(NB,),
    in_specs=[_spec((BLK, H), lambda i: (i, 0)),
              _spec((2, H), lambda i: (0, 0)),
              _spec((1, H), lambda i: (0, 0)),
              _spec((1, H), lambda i: (0, 0)),
              _spec((1, 1, BLK), lambda i: (i, 0, 0))],
    out_specs=[_spec((G, H), lambda i: (0, 0)),
               _spec((G, 1), lambda i: (0, 0))],
    out_shape=[jax.ShapeDtypeStruct((G, H), jnp.float32),
               jax.ShapeDtypeStruct((G, 1), jnp.float32)],
    scratch_shapes=[pltpu.VMEM((G, H), jnp.float32),
                    pltpu.VMEM((G, 1), jnp.float32)],
)

_fin = pl.pallas_call(
    _fin_body,
    in_specs=[_spec((G, H), lambda: (0, 0)),
              _spec((G, 1), lambda: (0, 0)),
              _spec((H, H), lambda: (0, 0)),
              _spec((1, H), lambda: (0, 0))],
    out_specs=_spec((G, H), lambda: (0, 0)),
    out_shape=jax.ShapeDtypeStruct((G, H), jnp.float32),
)


def kernel(x, edge_index, batch, W1_0, b1_0, W2_0, b2_0, g_0, be_0,
           W1_1, b1_1, W2_1, b2_1, g_1, be_1, W1_2, b1_2, W2_2, b2_2,
           g_2, be_2, Wp, bp):
    E = edge_index.shape[1]
    chunk = 2 * NS * CH
    epad = -(-E // chunk) * chunk
    pad = epad - E
    src = edge_index[0]
    dst = edge_index[1]
    ar = jnp.arange(pad, dtype=jnp.int32)
    src_p = jnp.concatenate([src, ar % N]).reshape(epad // 128, 1, 128)
    dst_p = jnp.concatenate([dst, N + ar % (ACC_ROWS - N)]
                            ).reshape(epad // 128, 1, 128)
    srcdst = jnp.concatenate([src_p, dst_p], axis=1)
    zer = jnp.zeros((ACC_ROWS, HH), jnp.float32)
    sc_agg = _make_sc_agg(epad)

    b1s = (b1_0.reshape(1, H), b1_1.reshape(1, H), b1_2.reshape(1, H))
    b2s = (b2_0.reshape(1, H), b2_1.reshape(1, H), b2_2.reshape(1, H))
    gs = (g_0.reshape(1, H), g_1.reshape(1, H), g_2.reshape(1, H))
    bes = (be_0.reshape(1, H), be_1.reshape(1, H), be_2.reshape(1, H))
    W2s = (W2_0, W2_1, W2_2)
    W1n = (W1_1, W1_2)

    t = _t0(x, W1_0)
    for l in range(3):
        agg2 = sc_agg(t, srcdst, zer)
        agg = agg2.reshape(2, N, HH)
        v, st = _k2(t, agg, b1s[l], W2s[l], b2s[l])
        if l < 2:
            t = _k3(v, st, gs[l], bes[l], W1n[l])
        else:
            ps, cnt = _k3p(v, st, gs[l], bes[l], batch.reshape(NB, 1, BLK))
    return _fin(ps, cnt, Wp, bp.reshape(1, H))
